# baseline (speedup 1.0000x reference)
import functools

import jax
import jax.numpy as jnp
from jax import lax
from jax.experimental import pallas as pl
from jax.experimental.pallas import tpu as pltpu
from jax.experimental.pallas import tpu_sc as plsc

N = 10000
E = 320000
IN_DIM = 128
HID = 64
OUT = 112

NC, NS, NW = 2, 16, 32
PERW = E // NW
RPT = N // NS
NDEG = 10240
RDEG = NDEG // NS

_mesh = plsc.VectorSubcoreMesh(core_axis_name="c", subcore_axis_name="s")
_sc_params = pltpu.CompilerParams(use_tc_tiling_on_sc=False)


def _sc_segsum(D, NBUF, BD, fused_cols=False):
    CHD = PERW // BD
    MAIN = (CHD // NBUF) * NBUF
    out_shape = (N, 2 * D) if fused_cols else (2 * N, D)

    @functools.partial(
        pl.kernel,
        out_type=jax.ShapeDtypeStruct(out_shape, jnp.float32),
        mesh=_mesh,
        scratch_types=[
            pltpu.VMEM((CHD, BD), jnp.int32),
            pltpu.VMEM((CHD, BD), jnp.int32),
            pltpu.VMEM((NBUF, BD, D), jnp.float32),
            pltpu.VMEM_SHARED((N, D), jnp.float32),
            pltpu.SemaphoreType.DMA((NBUF,)),
            pltpu.SemaphoreType.DMA((NBUF,)),
        ],
        compiler_params=_sc_params,
    )
    def f(srcs, dsts, table, zeros, out, src_v, dst_v, rows_v, acc, gsem, ssem):
        cid = lax.axis_index("c")
        sid = lax.axis_index("s")
        wid = cid * NS + sid
        zc = pltpu.async_copy(zeros, acc.at[pl.ds(sid * RPT, RPT)], gsem.at[0])
        sc_ = pltpu.async_copy(srcs.at[wid], src_v, gsem.at[1])
        dc = pltpu.async_copy(dsts.at[wid], dst_v, ssem.at[0])
        zc.wait()
        sc_.wait()
        dc.wait()
        plsc.subcore_barrier()

        def g_start(i, b):
            pltpu.async_copy(table.at[src_v.at[i]], rows_v.at[b], gsem.at[b])

        def g_wait(i, b):
            pltpu.make_async_copy(
                table.at[src_v.at[i]], rows_v.at[b], gsem.at[b]).wait()

        def s_start(i, b):
            pltpu.async_copy(
                rows_v.at[b], acc.at[dst_v.at[i]], ssem.at[b], add=True)

        def s_wait(i, b):
            pltpu.make_async_copy(
                rows_v.at[b], acc.at[dst_v.at[i]], ssem.at[b]).wait()

        for b in range(NBUF):
            g_start(b, b)

        def outer(k, carry):
            i0 = k * NBUF
            for b in range(NBUF):
                i = i0 + b
                g_wait(i, b)
                s_start(i, b)
                s_wait(i, b)

                @pl.when(i + NBUF < MAIN)
                def _():
                    g_start(i + NBUF, b)
            return carry

        lax.fori_loop(0, CHD // NBUF, outer, 0)
        for i in range(MAIN, CHD):
            g_start(i, 0)
            g_wait(i, 0)
            s_start(i, 0)
            s_wait(i, 0)
        plsc.subcore_barrier()
        if fused_cols:
            pltpu.sync_copy(acc.at[pl.ds(sid * RPT, RPT)],
                            out.at[pl.ds(sid * RPT, RPT), pl.ds(cid * D, D)])
        else:
            pltpu.sync_copy(acc.at[pl.ds(sid * RPT, RPT)],
                            out.at[pl.ds(cid * N + sid * RPT, RPT)])

    return f


_BDEG = 40
_CDEG = PERW // _BDEG


@functools.partial(
    pl.kernel,
    out_type=jax.ShapeDtypeStruct((2 * NDEG,), jnp.float32),
    mesh=_mesh,
    scratch_types=[
        pltpu.VMEM((_CDEG, _BDEG), jnp.int32),
        pltpu.VMEM((_BDEG,), jnp.float32),
        pltpu.VMEM_SHARED((NDEG,), jnp.float32),
        pltpu.SemaphoreType.DMA((2,)),
    ],
    compiler_params=_sc_params,
)
def _sc_deg(dsts, ones_hbm, zeros, out, dst_v, ones_v, acc, sem):
    cid = lax.axis_index("c")
    sid = lax.axis_index("s")
    wid = cid * NS + sid
    zc = pltpu.async_copy(zeros, acc.at[pl.ds(sid * RDEG, RDEG)], sem.at[0])
    dc = pltpu.async_copy(dsts.at[wid], dst_v, sem.at[1])
    zc.wait()
    oc = pltpu.async_copy(ones_hbm, ones_v, sem.at[0])
    dc.wait()
    oc.wait()
    plsc.subcore_barrier()

    def step(i, carry):
        pltpu.sync_copy(ones_v, acc.at[dst_v.at[i]], add=True)
        return carry

    lax.fori_loop(0, _CDEG, step, 0)
    plsc.subcore_barrier()
    pltpu.sync_copy(acc.at[pl.ds(sid * RDEG, RDEG)],
                    out.at[pl.ds(cid * NDEG + sid * RDEG, RDEG)])


def _tc_first_body(degp_ref, x_ref, w1_ref, dinv_ref, h1t_ref):
    deg = degp_ref[0, 0:N, :] + degp_ref[1, 0:N, :] + 1.0
    dinv = lax.rsqrt(deg)
    dinv_ref[...] = dinv
    h1 = jnp.dot(x_ref[...], w1_ref[...], preferred_element_type=jnp.float32)
    h1t_ref[...] = h1 * dinv


def _tc_mid_body(a_ref, ht_ref, dinv_ref, w_ref, b_ref, o_ref):
    dinv = dinv_ref[...]
    a = a_ref[:, 0:HID] + a_ref[:, HID:2 * HID]
    z = dinv * (a + ht_ref[...]) + b_ref[...]
    z = jnp.maximum(z, 0.0)
    o_ref[...] = dinv * jnp.dot(z, w_ref[...], preferred_element_type=jnp.float32)


def _tc_fin_body(a_ref, ht_ref, dinv_ref, b_ref, o_ref):
    o_ref[...] = (dinv_ref[...]
                  * (a_ref[0:N, 0:OUT] + a_ref[N:2 * N, 0:OUT]
                     + ht_ref[:, 0:OUT])
                  + b_ref[...])


_tc_first = pl.pallas_call(
    _tc_first_body,
    out_shape=(jax.ShapeDtypeStruct((N, 1), jnp.float32),
               jax.ShapeDtypeStruct((N, HID), jnp.float32)),
)


def _tc_mid(dout):
    return pl.pallas_call(
        _tc_mid_body,
        out_shape=jax.ShapeDtypeStruct((N, dout), jnp.float32),
    )


_tc_fin = pl.pallas_call(
    _tc_fin_body,
    out_shape=jax.ShapeDtypeStruct((N, OUT), jnp.float32),
)


def kernel(x, edge_index, W1, b1, W2, b2, W3, b3):
    src = edge_index[0].astype(jnp.int32)
    dst = edge_index[1].astype(jnp.int32)
    BE = 40
    srcs = src.reshape(NW, PERW // BE, BE)
    dsts = dst.reshape(NW, PERW // BE, BE)

    zeros_h = jnp.zeros((RPT, HID), jnp.float32)
    zeros_o = jnp.zeros((RPT, 128), jnp.float32)
    zeros_d = jnp.zeros((RDEG,), jnp.float32)
    ones_d = jnp.ones((_BDEG,), jnp.float32)

    degp = _sc_deg(dsts, ones_d, zeros_d)
    dinv, h1t = _tc_first(degp.reshape(2, NDEG, 1), x, W1)

    seg_h = _sc_segsum(HID, 4, BE, fused_cols=True)
    a1 = seg_h(srcs, dsts, h1t, zeros_h)
    h2t = _tc_mid(HID)(a1, h1t, dinv, W2, b1.reshape(1, -1))
    a2 = seg_h(srcs, dsts, h2t, zeros_h)
    W3p = jnp.pad(W3, ((0, 0), (0, 128 - OUT)))
    h3t = _tc_mid(128)(a2, h2t, dinv, W3p, b2.reshape(1, -1))
    a3 = _sc_segsum(128, 4, BE)(srcs, dsts, h3t, zeros_o)
    return _tc_fin(a3, h3t, dinv, b3.reshape(1, -1))


# speedup vs baseline: 1.1281x; 1.1281x over previous
import functools

import jax
import jax.numpy as jnp
from jax import lax
from jax.experimental import pallas as pl
from jax.experimental.pallas import tpu as pltpu
from jax.experimental.pallas import tpu_sc as plsc

N = 10000
E = 320000
IN_DIM = 128
HID = 64
OUT = 112

NC, NS, NW = 2, 16, 32
PERW = E // NW
RPT = N // NS
NDEG = 10240
RDEG = NDEG // NS

_mesh = plsc.VectorSubcoreMesh(core_axis_name="c", subcore_axis_name="s")
_sc_params = pltpu.CompilerParams(use_tc_tiling_on_sc=False)


def _sc_segsum(D, NBUF, BD, fused_cols=False):
    CHD = PERW // BD
    MAIN = (CHD // NBUF) * NBUF
    out_shape = (N, 2 * D) if fused_cols else (2 * N, D)

    @functools.partial(
        pl.kernel,
        out_type=jax.ShapeDtypeStruct(out_shape, jnp.float32),
        mesh=_mesh,
        scratch_types=[
            pltpu.VMEM((CHD, BD), jnp.int32),
            pltpu.VMEM((CHD, BD), jnp.int32),
            pltpu.VMEM((NBUF, BD, D), jnp.float32),
            pltpu.VMEM_SHARED((N, D), jnp.float32),
            pltpu.SemaphoreType.DMA((NBUF,)),
            pltpu.SemaphoreType.DMA((NBUF,)),
        ],
        compiler_params=_sc_params,
    )
    def f(srcs, dsts, table, zeros, out, src_v, dst_v, rows_v, acc, gsem, ssem):
        cid = lax.axis_index("c")
        sid = lax.axis_index("s")
        wid = cid * NS + sid
        zc = pltpu.async_copy(zeros, acc.at[pl.ds(sid * RPT, RPT)], gsem.at[0])
        sc_ = pltpu.async_copy(srcs.at[wid], src_v, gsem.at[1])
        dc = pltpu.async_copy(dsts.at[wid], dst_v, ssem.at[0])
        zc.wait()
        sc_.wait()
        dc.wait()
        plsc.subcore_barrier()

        def g_start(i, b):
            pltpu.async_copy(table.at[src_v.at[i]], rows_v.at[b], gsem.at[b])

        def g_wait(i, b):
            pltpu.make_async_copy(
                table.at[src_v.at[i]], rows_v.at[b], gsem.at[b]).wait()

        def s_start(i, b):
            pltpu.async_copy(
                rows_v.at[b], acc.at[dst_v.at[i]], ssem.at[b], add=True)

        def s_wait(i, b):
            pltpu.make_async_copy(
                rows_v.at[b], acc.at[dst_v.at[i]], ssem.at[b]).wait()

        for b in range(NBUF):
            g_start(b, b)

        def outer(k, carry):
            i0 = k * NBUF
            for b in range(NBUF):
                i = i0 + b
                g_wait(i, b)
                s_start(i, b)
                s_wait(i, b)

                @pl.when(i + NBUF < MAIN)
                def _():
                    g_start(i + NBUF, b)
            return carry

        lax.fori_loop(0, CHD // NBUF, outer, 0)
        for i in range(MAIN, CHD):
            g_start(i, 0)
            g_wait(i, 0)
            s_start(i, 0)
            s_wait(i, 0)
        plsc.subcore_barrier()
        if fused_cols:
            pltpu.sync_copy(acc.at[pl.ds(sid * RPT, RPT)],
                            out.at[pl.ds(sid * RPT, RPT), pl.ds(cid * D, D)])
        else:
            pltpu.sync_copy(acc.at[pl.ds(sid * RPT, RPT)],
                            out.at[pl.ds(cid * N + sid * RPT, RPT)])

    return f


_BDEG = 80
_CDEG = PERW // _BDEG


@functools.partial(
    pl.kernel,
    out_type=jax.ShapeDtypeStruct((2 * NDEG,), jnp.float32),
    mesh=_mesh,
    scratch_types=[
        pltpu.VMEM((_CDEG, _BDEG), jnp.int32),
        pltpu.VMEM((_BDEG,), jnp.float32),
        pltpu.VMEM_SHARED((NDEG,), jnp.float32),
        pltpu.SemaphoreType.DMA((2,)),
    ],
    compiler_params=_sc_params,
)
def _sc_deg(dsts, ones_hbm, zeros, out, dst_v, ones_v, acc, sem):
    cid = lax.axis_index("c")
    sid = lax.axis_index("s")
    wid = cid * NS + sid
    zc = pltpu.async_copy(zeros, acc.at[pl.ds(sid * RDEG, RDEG)], sem.at[0])
    dc = pltpu.async_copy(dsts.at[wid], dst_v, sem.at[1])
    zc.wait()
    oc = pltpu.async_copy(ones_hbm, ones_v, sem.at[0])
    dc.wait()
    oc.wait()
    plsc.subcore_barrier()

    def step(i, carry):
        pltpu.sync_copy(ones_v, acc.at[dst_v.at[i]], add=True)
        return carry

    lax.fori_loop(0, _CDEG, step, 0)
    plsc.subcore_barrier()
    pltpu.sync_copy(acc.at[pl.ds(sid * RDEG, RDEG)],
                    out.at[pl.ds(cid * NDEG + sid * RDEG, RDEG)])


def _tc_first_body(degp_ref, x_ref, w1_ref, dinv_ref, h1t_ref):
    deg = degp_ref[0, 0:N, :] + degp_ref[1, 0:N, :] + 1.0
    dinv = lax.rsqrt(deg)
    dinv_ref[...] = dinv
    h1 = jnp.dot(x_ref[...], w1_ref[...], preferred_element_type=jnp.float32)
    h1t_ref[...] = h1 * dinv


def _tc_mid_body(a_ref, ht_ref, dinv_ref, w_ref, b_ref, o_ref):
    dinv = dinv_ref[...]
    a = a_ref[:, 0:HID] + a_ref[:, HID:2 * HID]
    z = dinv * (a + ht_ref[...]) + b_ref[...]
    z = jnp.maximum(z, 0.0)
    o_ref[...] = dinv * jnp.dot(z, w_ref[...], preferred_element_type=jnp.float32)


def _tc_fin_body(a_ref, ht_ref, dinv_ref, b_ref, o_ref):
    o_ref[...] = (dinv_ref[...]
                  * (a_ref[0:N, 0:OUT] + a_ref[N:2 * N, 0:OUT]
                     + ht_ref[:, 0:OUT])
                  + b_ref[...])


_tc_first = pl.pallas_call(
    _tc_first_body,
    out_shape=(jax.ShapeDtypeStruct((N, 1), jnp.float32),
               jax.ShapeDtypeStruct((N, HID), jnp.float32)),
)


def _tc_mid(dout):
    return pl.pallas_call(
        _tc_mid_body,
        out_shape=jax.ShapeDtypeStruct((N, dout), jnp.float32),
    )


_tc_fin = pl.pallas_call(
    _tc_fin_body,
    out_shape=jax.ShapeDtypeStruct((N, OUT), jnp.float32),
)


def kernel(x, edge_index, W1, b1, W2, b2, W3, b3):
    src = edge_index[0].astype(jnp.int32)
    dst = edge_index[1].astype(jnp.int32)
    BE = 80
    srcs = src.reshape(NW, PERW // BE, BE)
    dsts = dst.reshape(NW, PERW // BE, BE)
    B3 = 40
    srcs3 = src.reshape(NW, PERW // B3, B3)
    dsts3 = dst.reshape(NW, PERW // B3, B3)

    zeros_h = jnp.zeros((RPT, HID), jnp.float32)
    zeros_o = jnp.zeros((RPT, 128), jnp.float32)
    zeros_d = jnp.zeros((RDEG,), jnp.float32)
    ones_d = jnp.ones((_BDEG,), jnp.float32)

    degp = _sc_deg(dsts, ones_d, zeros_d)
    dinv, h1t = _tc_first(degp.reshape(2, NDEG, 1), x, W1)

    seg_h = _sc_segsum(HID, 4, BE, fused_cols=True)
    a1 = seg_h(srcs, dsts, h1t, zeros_h)
    h2t = _tc_mid(HID)(a1, h1t, dinv, W2, b1.reshape(1, -1))
    a2 = seg_h(srcs, dsts, h2t, zeros_h)
    W3p = jnp.pad(W3, ((0, 0), (0, 128 - OUT)))
    h3t = _tc_mid(128)(a2, h2t, dinv, W3p, b2.reshape(1, -1))
    a3 = _sc_segsum(128, 4, B3)(srcs3, dsts3, h3t, zeros_o)
    return _tc_fin(a3, h3t, dinv, b3.reshape(1, -1))


# speedup vs baseline: 1.1380x; 1.0088x over previous
import functools

import jax
import jax.numpy as jnp
from jax import lax
from jax.experimental import pallas as pl
from jax.experimental.pallas import tpu as pltpu
from jax.experimental.pallas import tpu_sc as plsc

N = 10000
E = 320000
IN_DIM = 128
HID = 64
OUT = 112

NC, NS, NW = 2, 16, 32
PERW = E // NW
RPT = N // NS
NDEG = 10240
RDEG = NDEG // NS

_mesh = plsc.VectorSubcoreMesh(core_axis_name="c", subcore_axis_name="s")
_sc_params = pltpu.CompilerParams(use_tc_tiling_on_sc=False)


def _sc_segsum(D, NBUF, BD, fused_cols=False):
    CHD = PERW // BD
    MAIN = (CHD // NBUF) * NBUF
    out_shape = (N, 2 * D) if fused_cols else (2 * N, D)

    @functools.partial(
        pl.kernel,
        out_type=jax.ShapeDtypeStruct(out_shape, jnp.float32),
        mesh=_mesh,
        scratch_types=[
            pltpu.VMEM((CHD, BD), jnp.int32),
            pltpu.VMEM((CHD, BD), jnp.int32),
            pltpu.VMEM((NBUF, BD, D), jnp.float32),
            pltpu.VMEM_SHARED((N, D), jnp.float32),
            pltpu.SemaphoreType.DMA((NBUF,)),
            pltpu.SemaphoreType.DMA((NBUF,)),
        ],
        compiler_params=_sc_params,
    )
    def f(srcs, dsts, table, zeros, out, src_v, dst_v, rows_v, acc, gsem, ssem):
        cid = lax.axis_index("c")
        sid = lax.axis_index("s")
        wid = cid * NS + sid
        zc = pltpu.async_copy(zeros, acc.at[pl.ds(sid * RPT, RPT)], gsem.at[0])
        sc_ = pltpu.async_copy(srcs.at[wid], src_v, gsem.at[1])
        dc = pltpu.async_copy(dsts.at[wid], dst_v, ssem.at[0])
        zc.wait()
        sc_.wait()
        dc.wait()
        plsc.subcore_barrier()

        def g_start(i, b):
            pltpu.async_copy(table.at[src_v.at[i]], rows_v.at[b], gsem.at[b])

        def g_wait(i, b):
            pltpu.make_async_copy(
                table.at[src_v.at[i]], rows_v.at[b], gsem.at[b]).wait()

        def s_start(i, b):
            pltpu.async_copy(
                rows_v.at[b], acc.at[dst_v.at[i]], ssem.at[b], add=True)

        def s_wait(i, b):
            pltpu.make_async_copy(
                rows_v.at[b], acc.at[dst_v.at[i]], ssem.at[b]).wait()

        for b in range(NBUF):
            g_start(b, b)

        def outer(k, carry):
            i0 = k * NBUF
            for b in range(NBUF):
                i = i0 + b
                g_wait(i, b)
                s_start(i, b)
                s_wait(i, b)

                @pl.when(i + NBUF < MAIN)
                def _():
                    g_start(i + NBUF, b)
            return carry

        lax.fori_loop(0, CHD // NBUF, outer, 0)
        for i in range(MAIN, CHD):
            g_start(i, 0)
            g_wait(i, 0)
            s_start(i, 0)
            s_wait(i, 0)
        plsc.subcore_barrier()
        if fused_cols:
            pltpu.sync_copy(acc.at[pl.ds(sid * RPT, RPT)],
                            out.at[pl.ds(sid * RPT, RPT), pl.ds(cid * D, D)])
        else:
            pltpu.sync_copy(acc.at[pl.ds(sid * RPT, RPT)],
                            out.at[pl.ds(cid * N + sid * RPT, RPT)])

    return f


_BDEG = 80
_CDEG = PERW // _BDEG


@functools.partial(
    pl.kernel,
    out_type=jax.ShapeDtypeStruct((2 * NDEG,), jnp.float32),
    mesh=_mesh,
    scratch_types=[
        pltpu.VMEM((_CDEG, _BDEG), jnp.int32),
        pltpu.VMEM((_BDEG,), jnp.float32),
        pltpu.VMEM_SHARED((NDEG,), jnp.float32),
        pltpu.SemaphoreType.DMA((2,)),
    ],
    compiler_params=_sc_params,
)
def _sc_deg(dsts, ones_hbm, zeros, out, dst_v, ones_v, acc, sem):
    cid = lax.axis_index("c")
    sid = lax.axis_index("s")
    wid = cid * NS + sid
    zc = pltpu.async_copy(zeros, acc.at[pl.ds(sid * RDEG, RDEG)], sem.at[0])
    dc = pltpu.async_copy(dsts.at[wid], dst_v, sem.at[1])
    zc.wait()
    oc = pltpu.async_copy(ones_hbm, ones_v, sem.at[0])
    dc.wait()
    oc.wait()
    plsc.subcore_barrier()

    def step(i, carry):
        pltpu.sync_copy(ones_v, acc.at[dst_v.at[i]], add=True)
        return carry

    lax.fori_loop(0, _CDEG, step, 0)
    plsc.subcore_barrier()
    pltpu.sync_copy(acc.at[pl.ds(sid * RDEG, RDEG)],
                    out.at[pl.ds(cid * NDEG + sid * RDEG, RDEG)])


def _tc_first_body(degp_ref, x_ref, w1_ref, dinv_ref, h1t_ref):
    deg = degp_ref[0, 0:N, :] + degp_ref[1, 0:N, :] + 1.0
    dinv = lax.rsqrt(deg)
    dinv_ref[...] = dinv
    h1 = jnp.dot(x_ref[...], w1_ref[...], preferred_element_type=jnp.float32)
    h1t_ref[...] = h1 * dinv


def _tc_mid_body(a_ref, ht_ref, dinv_ref, w_ref, b_ref, o_ref):
    dinv = dinv_ref[...]
    a = a_ref[:, 0:HID] + a_ref[:, HID:2 * HID]
    z = dinv * (a + ht_ref[...]) + b_ref[...]
    z = jnp.maximum(z, 0.0)
    o_ref[...] = dinv * jnp.dot(z, w_ref[...], preferred_element_type=jnp.float32)


def _tc_fin_body(a_ref, ht_ref, dinv_ref, b_ref, o_ref):
    o_ref[...] = (dinv_ref[...]
                  * (a_ref[0:N, 0:OUT] + a_ref[N:2 * N, 0:OUT]
                     + ht_ref[:, 0:OUT])
                  + b_ref[...])


_tc_first = pl.pallas_call(
    _tc_first_body,
    out_shape=(jax.ShapeDtypeStruct((N, 1), jnp.float32),
               jax.ShapeDtypeStruct((N, HID), jnp.float32)),
)


def _tc_mid(dout):
    return pl.pallas_call(
        _tc_mid_body,
        out_shape=jax.ShapeDtypeStruct((N, dout), jnp.float32),
    )


_tc_fin = pl.pallas_call(
    _tc_fin_body,
    out_shape=jax.ShapeDtypeStruct((N, OUT), jnp.float32),
)


def kernel(x, edge_index, W1, b1, W2, b2, W3, b3):
    src = edge_index[0].astype(jnp.int32)
    dst = edge_index[1].astype(jnp.int32)
    BE = 80
    srcs = src.reshape(NW, PERW // BE, BE)
    dsts = dst.reshape(NW, PERW // BE, BE)
    B3 = 40
    srcs3 = src.reshape(NW, PERW // B3, B3)
    dsts3 = dst.reshape(NW, PERW // B3, B3)

    zeros_h = jnp.zeros((RPT, HID), jnp.float32)
    zeros_o = jnp.zeros((RPT, 128), jnp.float32)
    zeros_d = jnp.zeros((RDEG,), jnp.float32)
    ones_d = jnp.ones((_BDEG,), jnp.float32)

    degp = _sc_deg(dsts, ones_d, zeros_d)
    dinv, h1t = _tc_first(degp.reshape(2, NDEG, 1), x, W1)

    seg_h = _sc_segsum(HID, 4, BE, fused_cols=True)
    a1 = seg_h(srcs, dsts, h1t, zeros_h)
    h2t = _tc_mid(HID)(a1, h1t, dinv, W2, b1.reshape(1, -1))
    a2 = seg_h(srcs, dsts, h2t, zeros_h)
    W3p = jnp.pad(W3, ((0, 0), (0, 128 - OUT)))
    h3t = _tc_mid(128)(a2, h2t, dinv, W3p, b2.reshape(1, -1))
    a3 = _sc_segsum(128, 3, BE)(srcs, dsts, h3t, zeros_o)
    return _tc_fin(a3, h3t, dinv, b3.reshape(1, -1))


# speedup vs baseline: 1.1792x; 1.0362x over previous
import functools

import jax
import jax.numpy as jnp
from jax import lax
from jax.experimental import pallas as pl
from jax.experimental.pallas import tpu as pltpu
from jax.experimental.pallas import tpu_sc as plsc

N = 10000
E = 320000
IN_DIM = 128
HID = 64
OUT = 112

NC, NS, NW = 2, 16, 32
PERW = E // NW
RPT = N // NS
NDEG = 10240
RDEG = NDEG // NS

_mesh = plsc.VectorSubcoreMesh(core_axis_name="c", subcore_axis_name="s")
_sc_params = pltpu.CompilerParams(use_tc_tiling_on_sc=False)


def _sc_segsum(D, NBUF, BD, fused_cols=False):
    CHD = PERW // BD
    MAIN = (CHD // NBUF) * NBUF
    out_shape = (N, 2 * D) if fused_cols else (2 * N, D)

    @functools.partial(
        pl.kernel,
        out_type=jax.ShapeDtypeStruct(out_shape, jnp.float32),
        mesh=_mesh,
        scratch_types=[
            pltpu.VMEM((CHD, BD), jnp.int32),
            pltpu.VMEM((CHD, BD), jnp.int32),
            pltpu.VMEM((NBUF, BD, D), jnp.float32),
            pltpu.VMEM_SHARED((N, D), jnp.float32),
            pltpu.SemaphoreType.DMA((NBUF,)),
            pltpu.SemaphoreType.DMA((NBUF,)),
        ],
        compiler_params=_sc_params,
    )
    def f(edges, table, zeros, out, src_v, dst_v, rows_v, acc, gsem, ssem):
        cid = lax.axis_index("c")
        sid = lax.axis_index("s")
        wid = cid * NS + sid
        zc = pltpu.async_copy(zeros, acc.at[pl.ds(sid * RPT, RPT)], gsem.at[0])
        sc_ = pltpu.async_copy(edges.at[0, wid], src_v, gsem.at[1])
        dc = pltpu.async_copy(edges.at[1, wid], dst_v, ssem.at[0])
        zc.wait()
        sc_.wait()
        dc.wait()
        plsc.subcore_barrier()

        def g_start(i, b):
            pltpu.async_copy(table.at[src_v.at[i]], rows_v.at[b], gsem.at[b])

        def g_wait(i, b):
            pltpu.make_async_copy(
                table.at[src_v.at[i]], rows_v.at[b], gsem.at[b]).wait()

        def s_start(i, b):
            pltpu.async_copy(
                rows_v.at[b], acc.at[dst_v.at[i]], ssem.at[b], add=True)

        def s_wait(i, b):
            pltpu.make_async_copy(
                rows_v.at[b], acc.at[dst_v.at[i]], ssem.at[b]).wait()

        for b in range(NBUF):
            g_start(b, b)

        def outer(k, carry):
            i0 = k * NBUF
            for b in range(NBUF):
                i = i0 + b
                g_wait(i, b)
                s_start(i, b)
                s_wait(i, b)

                @pl.when(i + NBUF < MAIN)
                def _():
                    g_start(i + NBUF, b)
            return carry

        lax.fori_loop(0, CHD // NBUF, outer, 0)
        for i in range(MAIN, CHD):
            g_start(i, 0)
            g_wait(i, 0)
            s_start(i, 0)
            s_wait(i, 0)
        plsc.subcore_barrier()
        if fused_cols:
            pltpu.sync_copy(acc.at[pl.ds(sid * RPT, RPT)],
                            out.at[pl.ds(sid * RPT, RPT), pl.ds(cid * D, D)])
        else:
            pltpu.sync_copy(acc.at[pl.ds(sid * RPT, RPT)],
                            out.at[pl.ds(cid * N + sid * RPT, RPT)])

    return f


_BDEG = 80
_CDEG = PERW // _BDEG


@functools.partial(
    pl.kernel,
    out_type=jax.ShapeDtypeStruct((2 * NDEG,), jnp.float32),
    mesh=_mesh,
    scratch_types=[
        pltpu.VMEM((_CDEG, _BDEG), jnp.int32),
        pltpu.VMEM((_BDEG,), jnp.float32),
        pltpu.VMEM_SHARED((NDEG,), jnp.float32),
        pltpu.SemaphoreType.DMA((2,)),
    ],
    compiler_params=_sc_params,
)
def _sc_deg(edges, ones_hbm, zeros, out, dst_v, ones_v, acc, sem):
    cid = lax.axis_index("c")
    sid = lax.axis_index("s")
    wid = cid * NS + sid
    zc = pltpu.async_copy(zeros, acc.at[pl.ds(sid * RDEG, RDEG)], sem.at[0])
    dc = pltpu.async_copy(edges.at[1, wid], dst_v, sem.at[1])
    zc.wait()
    oc = pltpu.async_copy(ones_hbm, ones_v, sem.at[0])
    dc.wait()
    oc.wait()
    plsc.subcore_barrier()

    def step(i, carry):
        pltpu.sync_copy(ones_v, acc.at[dst_v.at[i]], add=True)
        return carry

    lax.fori_loop(0, _CDEG, step, 0)
    plsc.subcore_barrier()
    pltpu.sync_copy(acc.at[pl.ds(sid * RDEG, RDEG)],
                    out.at[pl.ds(cid * NDEG + sid * RDEG, RDEG)])


def _tc_first_body(degp_ref, x_ref, w1_ref, dinv_ref, h1t_ref):
    deg = degp_ref[0, 0:N, :] + degp_ref[1, 0:N, :] + 1.0
    dinv = lax.rsqrt(deg)
    dinv_ref[...] = dinv
    h1 = jnp.dot(x_ref[...], w1_ref[...], preferred_element_type=jnp.float32)
    h1t_ref[...] = h1 * dinv


def _tc_mid_body(a_ref, ht_ref, dinv_ref, w_ref, b_ref, o_ref):
    dinv = dinv_ref[...]
    a = a_ref[:, 0:HID] + a_ref[:, HID:2 * HID]
    z = dinv * (a + ht_ref[...]) + b_ref[...]
    z = jnp.maximum(z, 0.0)
    o_ref[...] = dinv * jnp.dot(z, w_ref[...], preferred_element_type=jnp.float32)


def _tc_fin_body(a_ref, ht_ref, dinv_ref, b_ref, o_ref):
    o_ref[...] = (dinv_ref[...]
                  * (a_ref[0:N, 0:OUT] + a_ref[N:2 * N, 0:OUT]
                     + ht_ref[:, 0:OUT])
                  + b_ref[...])


_tc_first = pl.pallas_call(
    _tc_first_body,
    out_shape=(jax.ShapeDtypeStruct((N, 1), jnp.float32),
               jax.ShapeDtypeStruct((N, HID), jnp.float32)),
)


def _tc_mid(dout):
    return pl.pallas_call(
        _tc_mid_body,
        out_shape=jax.ShapeDtypeStruct((N, dout), jnp.float32),
    )


_tc_fin = pl.pallas_call(
    _tc_fin_body,
    out_shape=jax.ShapeDtypeStruct((N, OUT), jnp.float32),
)


def kernel(x, edge_index, W1, b1, W2, b2, W3, b3):
    BE = 80
    edges = edge_index.astype(jnp.int32).reshape(2, NW, PERW // BE, BE)

    zeros_h = jnp.zeros((RPT, HID), jnp.float32)
    zeros_o = jnp.zeros((RPT, 128), jnp.float32)
    zeros_d = jnp.zeros((RDEG,), jnp.float32)
    ones_d = jnp.ones((_BDEG,), jnp.float32)

    degp = _sc_deg(edges, ones_d, zeros_d)
    dinv, h1t = _tc_first(degp.reshape(2, NDEG, 1), x, W1)

    seg_h = _sc_segsum(HID, 4, BE, fused_cols=True)
    a1 = seg_h(edges, h1t, zeros_h)
    h2t = _tc_mid(HID)(a1, h1t, dinv, W2, b1.reshape(1, -1))
    a2 = seg_h(edges, h2t, zeros_h)
    W3p = jnp.pad(W3, ((0, 0), (0, 128 - OUT)))
    h3t = _tc_mid(128)(a2, h2t, dinv, W3p, b2.reshape(1, -1))
    a3 = _sc_segsum(128, 3, BE)(edges, h3t, zeros_o)
    return _tc_fin(a3, h3t, dinv, b3.reshape(1, -1))
